# baseline (speedup 1.0000x reference)
import functools

import jax
import jax.numpy as jnp
from jax.experimental import pallas as pl
from jax.experimental.pallas import tpu as pltpu


def _conv3x3(src, w_ref, hs, HB, W, C, n_out):
    acc = jnp.zeros((HB * W, n_out), jnp.float32)
    for dh in range(3):
        for dw in range(3):
            lhs = src(hs + dh, dw).reshape(HB * W, C)
            wt = w_ref[(dh * 3 + dw) * C:(dh * 3 + dw + 1) * C, :]
            acc += jnp.dot(lhs, wt, preferred_element_type=jnp.float32)
    return acc


def _fused_kernel(x_ref, p_ref, w0_ref, b0_ref, w1_ref, b1_ref,
                  fw_ref, fb_ref, f0_ref, f1_ref, h_ref, *, H, W, C):
    H1, W1 = H // 2, W // 2

    CB = H // 4
    src0 = lambda h, dw: x_ref[0, h:h + CB, dw:dw + W, :]
    for c in range(4):
        hs = c * CB
        acc0 = _conv3x3(src0, w0_ref, hs, CB, W, C, f0_ref.shape[-1])
        f0_ref[0, hs:hs + CB] = (acc0 + b0_ref[...]).reshape(CB, W, -1)

    src1 = lambda h, dw: p_ref[0, h:h + H1, dw:dw + W1, :]
    acc1 = _conv3x3(src1, w1_ref, 0, H1, W1, C, f1_ref.shape[-1])
    acc1 = acc1 + b1_ref[...]
    f1_ref[0] = acc1.reshape(H1, W1, -1)

    pooled = jnp.sum(jnp.maximum(acc1, 0.0), axis=0, keepdims=True)
    h_ref[0] = (jnp.dot(pooled, fw_ref[...],
                        preferred_element_type=jnp.float32) + fb_ref[...])


def _repack_w(w):
    return jnp.transpose(w, (2, 3, 1, 0)).reshape(-1, w.shape[0])


def kernel(x, conv_w_0, conv_b_0, conv_w_1, conv_b_1, fc_w_t, fc_b):
    N, C0, H, W = x.shape
    C1 = conv_w_1.shape[0]
    V = fc_w_t.shape[1]
    H1, W1 = H // 2, W // 2

    xt = jnp.transpose(x, (0, 2, 3, 1)).astype(jnp.bfloat16)
    xt_pad = jnp.pad(xt, ((0, 0), (1, 1), (1, 1), (0, 0)))
    p = x.reshape(N, C0, H1, 2, W1, 2).mean(axis=(3, 5))
    p_pad = jnp.pad(jnp.transpose(p, (0, 2, 3, 1)).astype(jnp.bfloat16),
                    ((0, 0), (1, 1), (1, 1), (0, 0)))

    w0 = _repack_w(conv_w_0).astype(jnp.bfloat16)
    w1 = _repack_w(conv_w_1[:, :C0] + conv_w_1[:, C0:]
                   ).astype(jnp.bfloat16)

    body = functools.partial(_fused_kernel, H=H, W=W, C=C0)
    feat0, feat1, head = pl.pallas_call(
        body,
        out_shape=(
            jax.ShapeDtypeStruct((N, H, W, C0), jnp.float32),
            jax.ShapeDtypeStruct((N, H1, W1, C1), jnp.float32),
            jax.ShapeDtypeStruct((N, 1, V), jnp.float32),
        ),
        grid=(N,),
        in_specs=[
            pl.BlockSpec((1, H + 2, W + 2, C0), lambda n: (n, 0, 0, 0)),
            pl.BlockSpec((1, H1 + 2, W1 + 2, C0), lambda n: (n, 0, 0, 0)),
            pl.BlockSpec((9 * C0, C0), lambda n: (0, 0)),
            pl.BlockSpec((1, C0), lambda n: (0, 0)),
            pl.BlockSpec((9 * C0, C1), lambda n: (0, 0)),
            pl.BlockSpec((1, C1), lambda n: (0, 0)),
            pl.BlockSpec((C1, V), lambda n: (0, 0)),
            pl.BlockSpec((1, V), lambda n: (0, 0)),
        ],
        out_specs=(
            pl.BlockSpec((1, H, W, C0), lambda n: (n, 0, 0, 0)),
            pl.BlockSpec((1, H1, W1, C1), lambda n: (n, 0, 0, 0)),
            pl.BlockSpec((1, 1, V), lambda n: (n, 0, 0)),
        ),
        compiler_params=pltpu.CompilerParams(
            dimension_semantics=("parallel",)),
    )(xt_pad, p_pad, w0, conv_b_0.reshape(1, C0), w1, conv_b_1.reshape(1, C1),
      fc_w_t, fc_b.reshape(1, V))

    f0 = jnp.transpose(feat0, (0, 3, 1, 2))
    f1 = jnp.transpose(feat1, (0, 3, 1, 2))
    return [f1, f0], head.reshape(N, V)


# speedup vs baseline: 1.3031x; 1.3031x over previous
import functools

import jax
import jax.numpy as jnp
from jax.experimental import pallas as pl
from jax.experimental.pallas import tpu as pltpu


def _im2col(xb, HB, W, C):
    taps = []
    for dh in range(3):
        for dw in range(3):
            taps.append(xb[dh:dh + HB, dw:dw + W, :].reshape(HB * W, C))
    return jnp.concatenate(taps, axis=1)


def _fused_kernel(x_ref, p_ref, w0_ref, b0_ref, w1_ref, b1_ref,
                  fw_ref, fb_ref, f0_ref, f1_ref, h_ref, *, H, W, C):
    H1, W1 = H // 2, W // 2

    CB = H // 4
    for c in range(4):
        hs = c * CB
        xb = x_ref[0, hs:hs + CB + 2, :, :]
        xcol0 = _im2col(xb, CB, W, C)
        acc0 = jnp.dot(xcol0, w0_ref[...], preferred_element_type=jnp.float32)
        f0_ref[0, hs:hs + CB] = (acc0 + b0_ref[...]).reshape(CB, W, -1)

    pb = p_ref[0]
    xcol1 = _im2col(pb, H1, W1, C)
    acc1 = jnp.dot(xcol1, w1_ref[...], preferred_element_type=jnp.float32)
    acc1 = acc1 + b1_ref[...]
    f1_ref[0] = acc1.reshape(H1, W1, -1)

    pooled = jnp.sum(jnp.maximum(acc1, 0.0), axis=0, keepdims=True)
    h_ref[0] = (jnp.dot(pooled, fw_ref[...],
                        preferred_element_type=jnp.float32) + fb_ref[...])


def _repack_w(w):
    return jnp.transpose(w, (2, 3, 1, 0)).reshape(-1, w.shape[0])


def kernel(x, conv_w_0, conv_b_0, conv_w_1, conv_b_1, fc_w_t, fc_b):
    N, C0, H, W = x.shape
    C1 = conv_w_1.shape[0]
    V = fc_w_t.shape[1]
    H1, W1 = H // 2, W // 2

    xt = jnp.transpose(x, (0, 2, 3, 1)).astype(jnp.bfloat16)
    xt_pad = jnp.pad(xt, ((0, 0), (1, 1), (1, 1), (0, 0)))
    p = x.reshape(N, C0, H1, 2, W1, 2).mean(axis=(3, 5))
    p_pad = jnp.pad(jnp.transpose(p, (0, 2, 3, 1)).astype(jnp.bfloat16),
                    ((0, 0), (1, 1), (1, 1), (0, 0)))

    w0 = _repack_w(conv_w_0).astype(jnp.bfloat16)
    w1 = _repack_w(conv_w_1[:, :C0] + conv_w_1[:, C0:]
                   ).astype(jnp.bfloat16)

    body = functools.partial(_fused_kernel, H=H, W=W, C=C0)
    feat0, feat1, head = pl.pallas_call(
        body,
        out_shape=(
            jax.ShapeDtypeStruct((N, H, W, C0), jnp.float32),
            jax.ShapeDtypeStruct((N, H1, W1, C1), jnp.float32),
            jax.ShapeDtypeStruct((N, 1, V), jnp.float32),
        ),
        grid=(N,),
        in_specs=[
            pl.BlockSpec((1, H + 2, W + 2, C0), lambda n: (n, 0, 0, 0)),
            pl.BlockSpec((1, H1 + 2, W1 + 2, C0), lambda n: (n, 0, 0, 0)),
            pl.BlockSpec((9 * C0, C0), lambda n: (0, 0)),
            pl.BlockSpec((1, C0), lambda n: (0, 0)),
            pl.BlockSpec((9 * C0, C1), lambda n: (0, 0)),
            pl.BlockSpec((1, C1), lambda n: (0, 0)),
            pl.BlockSpec((C1, V), lambda n: (0, 0)),
            pl.BlockSpec((1, V), lambda n: (0, 0)),
        ],
        out_specs=(
            pl.BlockSpec((1, H, W, C0), lambda n: (n, 0, 0, 0)),
            pl.BlockSpec((1, H1, W1, C1), lambda n: (n, 0, 0, 0)),
            pl.BlockSpec((1, 1, V), lambda n: (n, 0, 0)),
        ),
        compiler_params=pltpu.CompilerParams(
            dimension_semantics=("parallel",)),
    )(xt_pad, p_pad, w0, conv_b_0.reshape(1, C0), w1, conv_b_1.reshape(1, C1),
      fc_w_t, fc_b.reshape(1, V))

    f0 = jnp.transpose(feat0, (0, 3, 1, 2))
    f1 = jnp.transpose(feat1, (0, 3, 1, 2))
    return [f1, f0], head.reshape(N, V)
